# baseline (speedup 1.0000x reference)
import functools

import jax
import jax.numpy as jnp
from jax import lax
from jax.experimental import pallas as pl
from jax.experimental.pallas import tpu as pltpu
from jax.experimental.pallas import tpu_sc as plsc

KK = 16
NB = 256
NROWS = 1_000_000
BATCH = 4096
NW = 32
NWORDS = NROWS // 4
W_MAIN = 7_808
W_TAIL = NWORDS - NW * W_MAIN
ITERS_MAIN = W_MAIN // 16
ITERS_TAIL = W_TAIL // 16
CHUNK_W = W_MAIN + W_TAIL
B_PER_W = BATCH // NW


def _sc_histogram_kernel():
    mesh = plsc.VectorSubcoreMesh(core_axis_name="c", subcore_axis_name="s",
                                  num_cores=2, num_subcores=16)

    @functools.partial(
        pl.kernel,
        out_type=[
            jax.ShapeDtypeStruct((NW, NB), jnp.int32),
            jax.ShapeDtypeStruct((BATCH,), jnp.int32),
        ],
        mesh=mesh,
        compiler_params=pltpu.CompilerParams(needs_layout_passes=False),
        scratch_types=[
            pltpu.VMEM((CHUNK_W,), jnp.int32),
            pltpu.VMEM((16, NB), jnp.int32),
            pltpu.VMEM((16, NB), jnp.int32),
            pltpu.VMEM((NB,), jnp.int32),
            pltpu.VMEM((B_PER_W,), jnp.int32),
            pltpu.VMEM((B_PER_W,), jnp.int32),
            pltpu.VMEM((B_PER_W,), jnp.int32),
            pltpu.VMEM((B_PER_W,), jnp.int32),
            pltpu.SemaphoreType.DMA,
            pltpu.SemaphoreType.DMA,
        ],
    )
    def sc_k(flat_hbm, bidx_hbm, hist_hbm, t_hbm,
             chunk, hista, histb, histrow, bidx_v, widx_v, g_v, t_v,
             sem_c, sem_g):
        wid = lax.axis_index("c") * 16 + lax.axis_index("s")
        lane = lax.iota(jnp.int32, 16)
        ones = jnp.full((16,), 1, jnp.int32)
        zeros = jnp.zeros((16,), jnp.int32)

        base = wid * W_MAIN
        cp = pltpu.async_copy(flat_hbm.at[pl.ds(base, W_MAIN)],
                              chunk.at[pl.ds(0, W_MAIN)], sem_c)

        pltpu.sync_copy(bidx_hbm.at[pl.ds(wid * B_PER_W, B_PER_W)], bidx_v)
        for j in range(B_PER_W // 16):
            sl = pl.ds(j * 16, 16)
            widx_v[sl] = bidx_v[sl] >> 2
        ga = pltpu.async_copy(flat_hbm.at[widx_v], g_v, sem_g)

        for l in range(16):
            for j in range(NB // 16):
                hista[l, pl.ds(j * 16, 16)] = zeros
                histb[l, pl.ds(j * 16, 16)] = zeros

        ga.wait()
        for j in range(B_PER_W // 16):
            sl = pl.ds(j * 16, 16)
            f = (g_v[sl] >> ((bidx_v[sl] & 3) * 8)) & 255
            t_v[sl] = (f & 15) * 16 + (f >> 4)
        pltpu.sync_copy(t_v, t_hbm.at[pl.ds(wid * B_PER_W, B_PER_W)])

        @pl.when(wid == NW - 1)
        def _():
            pltpu.sync_copy(flat_hbm.at[pl.ds(NW * W_MAIN, W_TAIL)],
                            chunk.at[pl.ds(W_MAIN, W_TAIL)])

        cp.wait()

        def scatter4(v):
            plsc.addupdate_scatter(hista, [lane, v & 255], ones)
            plsc.addupdate_scatter(histb, [lane, (v >> 8) & 255], ones)
            plsc.addupdate_scatter(hista, [lane, (v >> 16) & 255], ones)
            plsc.addupdate_scatter(histb, [lane, (v >> 24) & 255], ones)

        def hist_body(i, carry):
            scatter4(chunk[pl.ds(i * 16, 16)])
            return carry

        lax.fori_loop(0, ITERS_MAIN, hist_body, 0, unroll=4)

        @pl.when(wid == NW - 1)
        def _():
            for i in range(ITERS_MAIN, ITERS_MAIN + ITERS_TAIL):
                scatter4(chunk[pl.ds(i * 16, 16)])

        for j in range(NB // 16):
            sl = pl.ds(j * 16, 16)
            acc = hista[0, sl] + histb[0, sl]
            for l in range(1, 16):
                acc = acc + hista[l, sl] + histb[l, sl]
            histrow[sl] = acc
        pltpu.sync_copy(histrow, hist_hbm.at[wid])

    return sc_k


_SC_KERNEL_CACHE = []


def _sc_kernel():
    if not _SC_KERNEL_CACHE:
        _SC_KERNEL_CACHE.append(_sc_histogram_kernel())
    return _SC_KERNEL_CACHE[0]


C_BLK = 512
GRID = BATCH // C_BLK


def _tc1_body(x_ref, marg_ref, logz_ref):
    x = x_ref[...]
    m = jnp.max(x, axis=0, keepdims=True)
    e = jnp.exp(x - m)
    s = jnp.sum(e, axis=0, keepdims=True)
    en = e / s

    ri = lax.broadcasted_iota(jnp.int32, (KK, NB), 0)
    ci = lax.broadcasted_iota(jnp.int32, (KK, NB), 1)
    m0 = ((ci // KK) == ri).astype(jnp.float32)
    m1 = ((ci % KK) == ri).astype(jnp.float32)
    marg_ref[0:KK, :] = jnp.dot(m0, en, preferred_element_type=jnp.float32)
    marg_ref[KK:2 * KK, :] = jnp.dot(m1, en, preferred_element_type=jnp.float32)
    logz_ref[...] = m + jnp.log(s)


def _tc1_call(xt):
    return pl.pallas_call(
        _tc1_body,
        grid=(GRID,),
        in_specs=[pl.BlockSpec((NB, C_BLK), lambda g: (0, g))],
        out_specs=[
            pl.BlockSpec((2 * KK, C_BLK), lambda g: (0, g)),
            pl.BlockSpec((1, C_BLK), lambda g: (0, g)),
        ],
        out_shape=[
            jax.ShapeDtypeStruct((2 * KK, BATCH), jnp.float32),
            jax.ShapeDtypeStruct((1, BATCH), jnp.float32),
        ],
    )(xt)


def _tc2_body(x_ref, h_ref, t_ref, logz_ref, thr_ref, lc_ref, loss_ref):
    g = pl.program_id(0)
    jf = jnp.sum(h_ref[...], axis=0, keepdims=True)
    thrf = lax.rsqrt(jnp.sqrt(jf.astype(jnp.float32) + 1.0))

    @pl.when(g == 0)
    def _():
        thr_ref[...] = thrf
        ri = lax.broadcasted_iota(jnp.int32, (KK, NB), 0)
        ci = lax.broadcasted_iota(jnp.int32, (KK, NB), 1)
        jfb = jnp.broadcast_to(jf, (KK, NB))
        lc_ref[0, :] = jnp.sum(jnp.where((ci % KK) == ri, jfb, 0), axis=1)
        lc_ref[1, :] = jnp.sum(jnp.where((ci // KK) == ri, jfb, 0), axis=1)
        loss_ref[...] = jnp.zeros((1, 1), jnp.float32)

    x = x_ref[...]
    t = t_ref[0, 0, :].reshape(1, C_BLK)
    rows = lax.broadcasted_iota(jnp.int32, (NB, C_BLK), 0)
    mask = rows == t
    p = jnp.sum(jnp.where(mask, x, 0.0), axis=0, keepdims=True)
    w = jnp.dot(thrf, mask.astype(jnp.float32),
                preferred_element_type=jnp.float32)
    nll = logz_ref[...] - p
    part = jnp.sum(nll * w) * (1.0 / BATCH)
    loss_ref[...] += jnp.full((1, 1), 1.0, jnp.float32) * part


def _tc2_call(xt, hist_parts, t3, logz):
    return pl.pallas_call(
        _tc2_body,
        grid=(GRID,),
        in_specs=[
            pl.BlockSpec((NB, C_BLK), lambda g: (0, g)),
            pl.BlockSpec((NW, NB), lambda g: (0, 0)),
            pl.BlockSpec((1, 1, C_BLK), lambda g: (g, 0, 0)),
            pl.BlockSpec((1, C_BLK), lambda g: (0, g)),
        ],
        out_specs=[
            pl.BlockSpec((1, NB), lambda g: (0, 0)),
            pl.BlockSpec((2, KK), lambda g: (0, 0)),
            pl.BlockSpec((1, 1), lambda g: (0, 0)),
        ],
        out_shape=[
            jax.ShapeDtypeStruct((1, NB), jnp.float32),
            jax.ShapeDtypeStruct((2, KK), jnp.int32),
            jax.ShapeDtypeStruct((1, 1), jnp.float32),
        ],
    )(xt, hist_parts, t3, logz)


def kernel(Y, pred_log_prob, batch_idx):
    v = Y[:, 0:1] + Y[:, 1:2] * 16
    shifts = jnp.array([0, 8, 16, 24], jnp.int32)
    flat_packed = jnp.sum(v.reshape(NWORDS, 4) << shifts[None, :],
                          axis=1, dtype=jnp.int32)
    hist_parts, t = _sc_kernel()(flat_packed, batch_idx)
    xt = pred_log_prob.transpose(1, 2, 0).reshape(NB, BATCH)
    marg_t, logz = _tc1_call(xt)
    t3 = t.reshape(GRID, 1, C_BLK)
    thrf, lc, loss = _tc2_call(xt, hist_parts, t3, logz)
    marginals = marg_t.reshape(2, KK, BATCH).transpose(2, 0, 1)
    return (loss[0, 0], marginals, thrf.reshape(KK, KK), lc)


# speedup vs baseline: 1.1341x; 1.1341x over previous
import functools

import jax
import jax.numpy as jnp
from jax import lax
from jax.experimental import pallas as pl
from jax.experimental.pallas import tpu as pltpu
from jax.experimental.pallas import tpu_sc as plsc

KK = 16
NB = 256
NROWS = 1_000_000
BATCH = 4096
NW = 32
NWORDS = NROWS // 4
W_MAIN = 7_808
W_TAIL = NWORDS - NW * W_MAIN
ITERS_MAIN = W_MAIN // 16
ITERS_TAIL = W_TAIL // 16
CHUNK_W = W_MAIN + W_TAIL
B_PER_W = BATCH // NW


def _sc_histogram_kernel():
    mesh = plsc.VectorSubcoreMesh(core_axis_name="c", subcore_axis_name="s",
                                  num_cores=2, num_subcores=16)

    @functools.partial(
        pl.kernel,
        out_type=[
            jax.ShapeDtypeStruct((NW, NB), jnp.int32),
            jax.ShapeDtypeStruct((BATCH,), jnp.int32),
        ],
        mesh=mesh,
        compiler_params=pltpu.CompilerParams(needs_layout_passes=False),
        scratch_types=[
            pltpu.VMEM((CHUNK_W,), jnp.int32),
            pltpu.VMEM((16, NB), jnp.int32),
            pltpu.VMEM((16, NB), jnp.int32),
            pltpu.VMEM((NB,), jnp.int32),
            pltpu.VMEM((B_PER_W,), jnp.int32),
            pltpu.VMEM((B_PER_W,), jnp.int32),
            pltpu.VMEM((B_PER_W,), jnp.int32),
            pltpu.VMEM((B_PER_W,), jnp.int32),
            pltpu.SemaphoreType.DMA,
            pltpu.SemaphoreType.DMA,
        ],
    )
    def sc_k(flat_hbm, bidx_hbm, hist_hbm, t_hbm,
             chunk, hista, histb, histrow, bidx_v, widx_v, g_v, t_v,
             sem_c, sem_g):
        wid = lax.axis_index("c") * 16 + lax.axis_index("s")
        lane = lax.iota(jnp.int32, 16)
        ones = jnp.full((16,), 1, jnp.int32)
        zeros = jnp.zeros((16,), jnp.int32)

        base = wid * W_MAIN
        cp = pltpu.async_copy(flat_hbm.at[pl.ds(base, W_MAIN)],
                              chunk.at[pl.ds(0, W_MAIN)], sem_c)

        pltpu.sync_copy(bidx_hbm.at[pl.ds(wid * B_PER_W, B_PER_W)], bidx_v)
        for j in range(B_PER_W // 16):
            sl = pl.ds(j * 16, 16)
            widx_v[sl] = bidx_v[sl] >> 2
        ga = pltpu.async_copy(flat_hbm.at[widx_v], g_v, sem_g)

        for l in range(16):
            for j in range(NB // 16):
                hista[l, pl.ds(j * 16, 16)] = zeros
                histb[l, pl.ds(j * 16, 16)] = zeros

        ga.wait()
        for j in range(B_PER_W // 16):
            sl = pl.ds(j * 16, 16)
            f = (g_v[sl] >> ((bidx_v[sl] & 3) * 8)) & 255
            t_v[sl] = (f & 15) * 16 + (f >> 4)
        pltpu.sync_copy(t_v, t_hbm.at[pl.ds(wid * B_PER_W, B_PER_W)])

        @pl.when(wid == NW - 1)
        def _():
            pltpu.sync_copy(flat_hbm.at[pl.ds(NW * W_MAIN, W_TAIL)],
                            chunk.at[pl.ds(W_MAIN, W_TAIL)])

        cp.wait()

        def scatter4(v):
            plsc.addupdate_scatter(hista, [lane, v & 255], ones)
            plsc.addupdate_scatter(histb, [lane, (v >> 8) & 255], ones)
            plsc.addupdate_scatter(hista, [lane, (v >> 16) & 255], ones)
            plsc.addupdate_scatter(histb, [lane, (v >> 24) & 255], ones)

        def hist_body(i, carry):
            scatter4(chunk[pl.ds(i * 16, 16)])
            return carry

        lax.fori_loop(0, ITERS_MAIN, hist_body, 0, unroll=4)

        @pl.when(wid == NW - 1)
        def _():
            for i in range(ITERS_MAIN, ITERS_MAIN + ITERS_TAIL):
                scatter4(chunk[pl.ds(i * 16, 16)])

        for j in range(NB // 16):
            sl = pl.ds(j * 16, 16)
            acc = hista[0, sl] + histb[0, sl]
            for l in range(1, 16):
                acc = acc + hista[l, sl] + histb[l, sl]
            histrow[sl] = acc
        pltpu.sync_copy(histrow, hist_hbm.at[wid])

    return sc_k


_SC_KERNEL_CACHE = []


def _sc_kernel():
    if not _SC_KERNEL_CACHE:
        _SC_KERNEL_CACHE.append(_sc_histogram_kernel())
    return _SC_KERNEL_CACHE[0]


C_BLK = 512
GRID = BATCH // C_BLK


def _tc1_body(x_ref, marg_ref, logz_ref):
    x = x_ref[...]
    m = jnp.max(x, axis=0, keepdims=True)
    e = jnp.exp(x - m)
    s = jnp.sum(e, axis=0, keepdims=True)
    en = e / s

    ri = lax.broadcasted_iota(jnp.int32, (KK, NB), 0)
    ci = lax.broadcasted_iota(jnp.int32, (KK, NB), 1)
    m0 = ((ci // KK) == ri).astype(jnp.float32)
    m1 = ((ci % KK) == ri).astype(jnp.float32)
    marg_ref[0:KK, :] = jnp.dot(m0, en, preferred_element_type=jnp.float32)
    marg_ref[KK:2 * KK, :] = jnp.dot(m1, en, preferred_element_type=jnp.float32)
    logz_ref[...] = m + jnp.log(s)


def _tc1_call(xt):
    return pl.pallas_call(
        _tc1_body,
        grid=(GRID,),
        in_specs=[pl.BlockSpec((NB, C_BLK), lambda g: (0, g))],
        out_specs=[
            pl.BlockSpec((2 * KK, C_BLK), lambda g: (0, g)),
            pl.BlockSpec((1, C_BLK), lambda g: (0, g)),
        ],
        out_shape=[
            jax.ShapeDtypeStruct((2 * KK, BATCH), jnp.float32),
            jax.ShapeDtypeStruct((1, BATCH), jnp.float32),
        ],
    )(xt)


def _tc2_body(x_ref, h_ref, t_ref, logz_ref, thr_ref, lc_ref, loss_ref):
    g = pl.program_id(0)
    jf = jnp.sum(h_ref[...], axis=0, keepdims=True)
    thrf = lax.rsqrt(jnp.sqrt(jf.astype(jnp.float32) + 1.0))

    @pl.when(g == 0)
    def _():
        thr_ref[...] = thrf
        ri = lax.broadcasted_iota(jnp.int32, (KK, NB), 0)
        ci = lax.broadcasted_iota(jnp.int32, (KK, NB), 1)
        jfb = jnp.broadcast_to(jf, (KK, NB))
        lc_ref[0, :] = jnp.sum(jnp.where((ci % KK) == ri, jfb, 0), axis=1)
        lc_ref[1, :] = jnp.sum(jnp.where((ci // KK) == ri, jfb, 0), axis=1)
        loss_ref[...] = jnp.zeros((1, 1), jnp.float32)

    x = x_ref[...]
    t = t_ref[0, 0, :].reshape(1, C_BLK)
    rows = lax.broadcasted_iota(jnp.int32, (NB, C_BLK), 0)
    mask = rows == t
    p = jnp.sum(jnp.where(mask, x, 0.0), axis=0, keepdims=True)
    w = jnp.dot(thrf, mask.astype(jnp.float32),
                preferred_element_type=jnp.float32)
    nll = logz_ref[...] - p
    part = jnp.sum(nll * w) * (1.0 / BATCH)
    loss_ref[...] += jnp.full((1, 1), 1.0, jnp.float32) * part


def _tc2_call(xt, hist_parts, t3, logz):
    return pl.pallas_call(
        _tc2_body,
        grid=(GRID,),
        in_specs=[
            pl.BlockSpec((NB, C_BLK), lambda g: (0, g)),
            pl.BlockSpec((NW, NB), lambda g: (0, 0)),
            pl.BlockSpec((1, 1, C_BLK), lambda g: (g, 0, 0)),
            pl.BlockSpec((1, C_BLK), lambda g: (0, g)),
        ],
        out_specs=[
            pl.BlockSpec((1, NB), lambda g: (0, 0)),
            pl.BlockSpec((2, KK), lambda g: (0, 0)),
            pl.BlockSpec((1, 1), lambda g: (0, 0)),
        ],
        out_shape=[
            jax.ShapeDtypeStruct((1, NB), jnp.float32),
            jax.ShapeDtypeStruct((2, KK), jnp.int32),
            jax.ShapeDtypeStruct((1, 1), jnp.float32),
        ],
    )(xt, hist_parts, t3, logz)


def kernel(Y, pred_log_prob, batch_idx):
    f = (Y[:, 0:1] + Y[:, 1:2] * 16).reshape(NROWS)
    flat_packed = (f[0::4] | (f[1::4] << 8) | (f[2::4] << 16)
                   | (f[3::4] << 24))
    hist_parts, t = _sc_kernel()(flat_packed, batch_idx)
    xt = pred_log_prob.transpose(1, 2, 0).reshape(NB, BATCH)
    marg_t, logz = _tc1_call(xt)
    t3 = t.reshape(GRID, 1, C_BLK)
    thrf, lc, loss = _tc2_call(xt, hist_parts, t3, logz)
    marginals = marg_t.reshape(2, KK, BATCH).transpose(2, 0, 1)
    return (loss[0, 0], marginals, thrf.reshape(KK, KK), lc)


# speedup vs baseline: 2.5137x; 2.2165x over previous
import functools

import jax
import jax.numpy as jnp
from jax import lax
from jax.experimental import pallas as pl
from jax.experimental.pallas import tpu as pltpu
from jax.experimental.pallas import tpu_sc as plsc

KK = 16
NB = 256
NROWS = 1_000_000
HALF = NROWS // 2
BATCH = 4096
NW = 32
W_MAIN = 15_600
W_TAIL = HALF - NW * W_MAIN
ITERS_MAIN = W_MAIN // 16
ITERS_TAIL = W_TAIL // 16
CHUNK_W = W_MAIN + W_TAIL
B_PER_W = BATCH // NW


def _make_sc_kernel(do_batch):
    mesh = plsc.VectorSubcoreMesh(core_axis_name="c", subcore_axis_name="s",
                                  num_cores=2, num_subcores=16)
    out_type = [jax.ShapeDtypeStruct((NW, NB), jnp.int32)]
    scratch = [
        pltpu.VMEM((CHUNK_W,), jnp.int32),
        pltpu.VMEM((16, NB), jnp.int32),
        pltpu.VMEM((16, NB), jnp.int32),
        pltpu.VMEM((NB,), jnp.int32),
        pltpu.SemaphoreType.DMA,
    ]
    if do_batch:
        out_type.append(jax.ShapeDtypeStruct((BATCH,), jnp.int32))
        scratch += [
            pltpu.VMEM((B_PER_W,), jnp.int32),
            pltpu.VMEM((B_PER_W,), jnp.int32),
            pltpu.VMEM((B_PER_W,), jnp.int32),
            pltpu.VMEM((B_PER_W,), jnp.int32),
            pltpu.VMEM((B_PER_W,), jnp.int32),
            pltpu.VMEM((B_PER_W,), jnp.int32),
            pltpu.SemaphoreType.DMA,
        ]

    def body(refs):
        if do_batch:
            (flat_hbm, flata_hbm, bidx_hbm, hist_hbm, t_hbm,
             chunk, hista, histb, histrow, sem_c,
             bidx_v, ia_v, ib_v, ga_v, gb_v, t_v, sem_g) = refs
        else:
            (flat_hbm, hist_hbm,
             chunk, hista, histb, histrow, sem_c) = refs
        wid = lax.axis_index("c") * 16 + lax.axis_index("s")
        lane = lax.iota(jnp.int32, 16)
        ones = jnp.full((16,), 1, jnp.int32)
        zeros = jnp.zeros((16,), jnp.int32)

        base = wid * W_MAIN
        cp = pltpu.async_copy(flat_hbm.at[pl.ds(base, W_MAIN)],
                              chunk.at[pl.ds(0, W_MAIN)], sem_c)

        if do_batch:
            pltpu.sync_copy(bidx_hbm.at[pl.ds(wid * B_PER_W, B_PER_W)],
                            bidx_v)
            for j in range(B_PER_W // 16):
                sl = pl.ds(j * 16, 16)
                b = bidx_v[sl]
                ia_v[sl] = jnp.minimum(b, HALF - 1)
                ib_v[sl] = jnp.maximum(b - HALF, 0)
            g0 = pltpu.async_copy(flata_hbm.at[ia_v], ga_v, sem_g)
            g1 = pltpu.async_copy(flat_hbm.at[ib_v], gb_v, sem_g)
            g0.wait()
            g1.wait()
            for j in range(B_PER_W // 16):
                sl = pl.ds(j * 16, 16)
                f = jnp.where(bidx_v[sl] < HALF, ga_v[sl], gb_v[sl])
                t_v[sl] = (f & 15) * 16 + (f >> 4)
            pltpu.sync_copy(t_v, t_hbm.at[pl.ds(wid * B_PER_W, B_PER_W)])

        for l in range(16):
            for j in range(NB // 16):
                hista[l, pl.ds(j * 16, 16)] = zeros
                histb[l, pl.ds(j * 16, 16)] = zeros

        @pl.when(wid == NW - 1)
        def _():
            pltpu.sync_copy(flat_hbm.at[pl.ds(NW * W_MAIN, W_TAIL)],
                            chunk.at[pl.ds(W_MAIN, W_TAIL)])

        cp.wait()

        def scatter2(i):
            plsc.addupdate_scatter(hista, [lane, chunk[pl.ds(i * 32, 16)]],
                                   ones)
            plsc.addupdate_scatter(histb,
                                   [lane, chunk[pl.ds(i * 32 + 16, 16)]],
                                   ones)

        def hist_body(i, carry):
            scatter2(i)
            return carry

        lax.fori_loop(0, ITERS_MAIN // 2, hist_body, 0, unroll=4)
        plsc.addupdate_scatter(
            hista, [lane, chunk[pl.ds((ITERS_MAIN - 1) * 16, 16)]], ones)

        @pl.when(wid == NW - 1)
        def _():
            for i in range(ITERS_TAIL // 2):
                plsc.addupdate_scatter(
                    hista,
                    [lane, chunk[pl.ds(W_MAIN + i * 32, 16)]], ones)
                plsc.addupdate_scatter(
                    histb,
                    [lane, chunk[pl.ds(W_MAIN + i * 32 + 16, 16)]], ones)

        for j in range(NB // 16):
            sl = pl.ds(j * 16, 16)
            acc = hista[0, sl] + histb[0, sl]
            for l in range(1, 16):
                acc = acc + hista[l, sl] + histb[l, sl]
            histrow[sl] = acc
        pltpu.sync_copy(histrow, hist_hbm.at[wid])

    if do_batch:
        @functools.partial(pl.kernel, out_type=out_type, mesh=mesh,
                           compiler_params=pltpu.CompilerParams(
                               needs_layout_passes=False),
                           scratch_types=scratch)
        def sc_k(*refs):
            body(refs)
    else:
        @functools.partial(pl.kernel, out_type=out_type, mesh=mesh,
                           compiler_params=pltpu.CompilerParams(
                               needs_layout_passes=False),
                           scratch_types=scratch)
        def sc_k(*refs):
            body(refs)
    return sc_k


_SC_CACHE = {}


def _sc_kernel(do_batch):
    if do_batch not in _SC_CACHE:
        _SC_CACHE[do_batch] = _make_sc_kernel(do_batch)
    return _SC_CACHE[do_batch]


C_BLK = 512
GRID = BATCH // C_BLK


def _tc1_body(x_ref, marg_ref, logz_ref):
    x = x_ref[...]
    m = jnp.max(x, axis=0, keepdims=True)
    e = jnp.exp(x - m)
    s = jnp.sum(e, axis=0, keepdims=True)
    en = e / s

    ri = lax.broadcasted_iota(jnp.int32, (KK, NB), 0)
    ci = lax.broadcasted_iota(jnp.int32, (KK, NB), 1)
    m0 = ((ci // KK) == ri).astype(jnp.float32)
    m1 = ((ci % KK) == ri).astype(jnp.float32)
    marg_ref[0:KK, :] = jnp.dot(m0, en, preferred_element_type=jnp.float32)
    marg_ref[KK:2 * KK, :] = jnp.dot(m1, en, preferred_element_type=jnp.float32)
    logz_ref[...] = m + jnp.log(s)


def _tc1_call(xt):
    return pl.pallas_call(
        _tc1_body,
        grid=(GRID,),
        in_specs=[pl.BlockSpec((NB, C_BLK), lambda g: (0, g))],
        out_specs=[
            pl.BlockSpec((2 * KK, C_BLK), lambda g: (0, g)),
            pl.BlockSpec((1, C_BLK), lambda g: (0, g)),
        ],
        out_shape=[
            jax.ShapeDtypeStruct((2 * KK, BATCH), jnp.float32),
            jax.ShapeDtypeStruct((1, BATCH), jnp.float32),
        ],
    )(xt)


def _tc2_body(x_ref, ha_ref, hb_ref, t_ref, logz_ref,
              thr_ref, lc_ref, loss_ref):
    g = pl.program_id(0)
    jf = (jnp.sum(ha_ref[...], axis=0, keepdims=True)
          + jnp.sum(hb_ref[...], axis=0, keepdims=True))
    thrf = lax.rsqrt(jnp.sqrt(jf.astype(jnp.float32) + 1.0))

    @pl.when(g == 0)
    def _():
        thr_ref[...] = thrf
        ri = lax.broadcasted_iota(jnp.int32, (KK, NB), 0)
        ci = lax.broadcasted_iota(jnp.int32, (KK, NB), 1)
        jfb = jnp.broadcast_to(jf, (KK, NB))
        lc_ref[0, :] = jnp.sum(jnp.where((ci % KK) == ri, jfb, 0), axis=1)
        lc_ref[1, :] = jnp.sum(jnp.where((ci // KK) == ri, jfb, 0), axis=1)
        loss_ref[...] = jnp.zeros((1, 1), jnp.float32)

    x = x_ref[...]
    t = t_ref[0, 0, :].reshape(1, C_BLK)
    rows = lax.broadcasted_iota(jnp.int32, (NB, C_BLK), 0)
    mask = rows == t
    p = jnp.sum(jnp.where(mask, x, 0.0), axis=0, keepdims=True)
    w = jnp.dot(thrf, mask.astype(jnp.float32),
                preferred_element_type=jnp.float32)
    nll = logz_ref[...] - p
    part = jnp.sum(nll * w) * (1.0 / BATCH)
    loss_ref[...] += jnp.full((1, 1), 1.0, jnp.float32) * part


def _tc2_call(xt, hist_a, hist_b, t3, logz):
    return pl.pallas_call(
        _tc2_body,
        grid=(GRID,),
        in_specs=[
            pl.BlockSpec((NB, C_BLK), lambda g: (0, g)),
            pl.BlockSpec((NW, NB), lambda g: (0, 0)),
            pl.BlockSpec((NW, NB), lambda g: (0, 0)),
            pl.BlockSpec((1, 1, C_BLK), lambda g: (g, 0, 0)),
            pl.BlockSpec((1, C_BLK), lambda g: (0, g)),
        ],
        out_specs=[
            pl.BlockSpec((1, NB), lambda g: (0, 0)),
            pl.BlockSpec((2, KK), lambda g: (0, 0)),
            pl.BlockSpec((1, 1), lambda g: (0, 0)),
        ],
        out_shape=[
            jax.ShapeDtypeStruct((1, NB), jnp.float32),
            jax.ShapeDtypeStruct((2, KK), jnp.int32),
            jax.ShapeDtypeStruct((1, 1), jnp.float32),
        ],
    )(xt, hist_a, hist_b, t3, logz)


def kernel(Y, pred_log_prob, batch_idx):
    fa = (Y[:HALF, 0:1] + Y[:HALF, 1:2] * 16).reshape(HALF)
    fa, y2 = lax.optimization_barrier((fa, Y))
    fb = (y2[HALF:, 0:1] + y2[HALF:, 1:2] * 16).reshape(HALF)

    hist_a = _sc_kernel(False)(fa)[0]
    hist_b, t = _sc_kernel(True)(fb, fa, batch_idx)

    xt = pred_log_prob.transpose(1, 2, 0).reshape(NB, BATCH)
    marg_t, logz = _tc1_call(xt)
    t3 = t.reshape(GRID, 1, C_BLK)
    thrf, lc, loss = _tc2_call(xt, hist_a, hist_b, t3, logz)
    marginals = marg_t.reshape(2, KK, BATCH).transpose(2, 0, 1)
    return (loss[0, 0], marginals, thrf.reshape(KK, KK), lc)


# speedup vs baseline: 3.2710x; 1.3013x over previous
import functools

import jax
import jax.numpy as jnp
from jax import lax
from jax.experimental import pallas as pl
from jax.experimental.pallas import tpu as pltpu
from jax.experimental.pallas import tpu_sc as plsc

KK = 16
NB = 256
NROWS = 1_000_000
BATCH = 4096
NW = 32
ROWS_MAIN = 31_248
ROWS_TAIL = NROWS - NW * ROWS_MAIN
ITERS_MAIN = ROWS_MAIN // 16
ITERS_TAIL = ROWS_TAIL // 16
CHUNK_W = ROWS_MAIN + ROWS_TAIL
B_PER_W = BATCH // NW


def _sc_histogram_kernel():
    mesh = plsc.VectorSubcoreMesh(core_axis_name="c", subcore_axis_name="s",
                                  num_cores=2, num_subcores=16)

    @functools.partial(
        pl.kernel,
        out_type=[
            jax.ShapeDtypeStruct((NW, NB), jnp.int32),
            jax.ShapeDtypeStruct((BATCH,), jnp.int32),
        ],
        mesh=mesh,
        compiler_params=pltpu.CompilerParams(needs_layout_passes=False),
        scratch_types=[
            pltpu.VMEM((CHUNK_W,), jnp.int32),
            pltpu.VMEM((16, NB), jnp.int32),
            pltpu.VMEM((NB,), jnp.int32),
            pltpu.VMEM((B_PER_W,), jnp.int32),
            pltpu.VMEM((B_PER_W,), jnp.int32),
            pltpu.VMEM((B_PER_W,), jnp.int32),
            pltpu.SemaphoreType.DMA,
            pltpu.SemaphoreType.DMA,
        ],
    )
    def sc_k(flat_hbm, bidx_hbm, hist_hbm, t_hbm,
             chunk, hist, histrow, bidx_v, g_v, t_v, sem_c, sem_g):
        wid = lax.axis_index("c") * 16 + lax.axis_index("s")
        lane = lax.iota(jnp.int32, 16)
        ones = jnp.full((16,), 1, jnp.int32)
        zeros = jnp.zeros((16,), jnp.int32)

        base = wid * ROWS_MAIN
        cp = pltpu.async_copy(flat_hbm.at[pl.ds(base, ROWS_MAIN)],
                              chunk.at[pl.ds(0, ROWS_MAIN)], sem_c)

        pltpu.sync_copy(bidx_hbm.at[pl.ds(wid * B_PER_W, B_PER_W)], bidx_v)
        ga = pltpu.async_copy(flat_hbm.at[bidx_v], g_v, sem_g)

        for l in range(16):
            for j in range(NB // 16):
                hist[l, pl.ds(j * 16, 16)] = zeros

        ga.wait()
        for j in range(B_PER_W // 16):
            sl = pl.ds(j * 16, 16)
            f = g_v[sl]
            t_v[sl] = (f & 15) * 16 + (f >> 4)
        pltpu.sync_copy(t_v, t_hbm.at[pl.ds(wid * B_PER_W, B_PER_W)])

        @pl.when(wid == NW - 1)
        def _():
            pltpu.sync_copy(flat_hbm.at[pl.ds(NW * ROWS_MAIN, ROWS_TAIL)],
                            chunk.at[pl.ds(ROWS_MAIN, ROWS_TAIL)])

        cp.wait()

        def hist_body(i, carry):
            flat = chunk[pl.ds(i * 16, 16)]
            plsc.addupdate_scatter(hist, [lane, flat], ones)
            return carry

        lax.fori_loop(0, ITERS_MAIN, hist_body, 0, unroll=4)

        @pl.when(wid == NW - 1)
        def _():
            for i in range(ITERS_MAIN, ITERS_MAIN + ITERS_TAIL):
                flat = chunk[pl.ds(i * 16, 16)]
                plsc.addupdate_scatter(hist, [lane, flat], ones)

        for j in range(NB // 16):
            acc = hist[0, pl.ds(j * 16, 16)]
            for l in range(1, 16):
                acc = acc + hist[l, pl.ds(j * 16, 16)]
            histrow[pl.ds(j * 16, 16)] = acc
        pltpu.sync_copy(histrow, hist_hbm.at[wid])

    return sc_k


_SC_KERNEL_CACHE = []


def _sc_kernel():
    if not _SC_KERNEL_CACHE:
        _SC_KERNEL_CACHE.append(_sc_histogram_kernel())
    return _SC_KERNEL_CACHE[0]


C_BLK = 512
GRID = BATCH // C_BLK


def _tc1_body(x_ref, marg_ref, logz_ref):
    x = x_ref[...]
    m = jnp.max(x, axis=0, keepdims=True)
    e = jnp.exp(x - m)
    s = jnp.sum(e, axis=0, keepdims=True)
    en = e / s

    ri = lax.broadcasted_iota(jnp.int32, (KK, NB), 0)
    ci = lax.broadcasted_iota(jnp.int32, (KK, NB), 1)
    m0 = ((ci // KK) == ri).astype(jnp.float32)
    m1 = ((ci % KK) == ri).astype(jnp.float32)
    marg_ref[0:KK, :] = jnp.dot(m0, en, preferred_element_type=jnp.float32)
    marg_ref[KK:2 * KK, :] = jnp.dot(m1, en, preferred_element_type=jnp.float32)
    logz_ref[...] = m + jnp.log(s)


def _tc1_call(xt):
    return pl.pallas_call(
        _tc1_body,
        grid=(GRID,),
        in_specs=[pl.BlockSpec((NB, C_BLK), lambda g: (0, g))],
        out_specs=[
            pl.BlockSpec((2 * KK, C_BLK), lambda g: (0, g)),
            pl.BlockSpec((1, C_BLK), lambda g: (0, g)),
        ],
        out_shape=[
            jax.ShapeDtypeStruct((2 * KK, BATCH), jnp.float32),
            jax.ShapeDtypeStruct((1, BATCH), jnp.float32),
        ],
    )(xt)


def _tc2_body(x_ref, h_ref, t_ref, logz_ref, thr_ref, lc_ref, loss_ref):
    g = pl.program_id(0)
    jf = jnp.sum(h_ref[...], axis=0, keepdims=True)
    thrf = lax.rsqrt(jnp.sqrt(jf.astype(jnp.float32) + 1.0))

    @pl.when(g == 0)
    def _():
        thr_ref[...] = thrf
        ri = lax.broadcasted_iota(jnp.int32, (KK, NB), 0)
        ci = lax.broadcasted_iota(jnp.int32, (KK, NB), 1)
        jfb = jnp.broadcast_to(jf, (KK, NB))
        lc_ref[0, :] = jnp.sum(jnp.where((ci % KK) == ri, jfb, 0), axis=1)
        lc_ref[1, :] = jnp.sum(jnp.where((ci // KK) == ri, jfb, 0), axis=1)
        loss_ref[...] = jnp.zeros((1, 1), jnp.float32)

    x = x_ref[...]
    t = t_ref[0, 0, :].reshape(1, C_BLK)
    rows = lax.broadcasted_iota(jnp.int32, (NB, C_BLK), 0)
    mask = rows == t
    p = jnp.sum(jnp.where(mask, x, 0.0), axis=0, keepdims=True)
    w = jnp.dot(thrf, mask.astype(jnp.float32),
                preferred_element_type=jnp.float32)
    nll = logz_ref[...] - p
    part = jnp.sum(nll * w) * (1.0 / BATCH)
    loss_ref[...] += jnp.full((1, 1), 1.0, jnp.float32) * part


def _tc2_call(xt, hist_parts, t3, logz):
    return pl.pallas_call(
        _tc2_body,
        grid=(GRID,),
        in_specs=[
            pl.BlockSpec((NB, C_BLK), lambda g: (0, g)),
            pl.BlockSpec((NW, NB), lambda g: (0, 0)),
            pl.BlockSpec((1, 1, C_BLK), lambda g: (g, 0, 0)),
            pl.BlockSpec((1, C_BLK), lambda g: (0, g)),
        ],
        out_specs=[
            pl.BlockSpec((1, NB), lambda g: (0, 0)),
            pl.BlockSpec((2, KK), lambda g: (0, 0)),
            pl.BlockSpec((1, 1), lambda g: (0, 0)),
        ],
        out_shape=[
            jax.ShapeDtypeStruct((1, NB), jnp.float32),
            jax.ShapeDtypeStruct((2, KK), jnp.int32),
            jax.ShapeDtypeStruct((1, 1), jnp.float32),
        ],
    )(xt, hist_parts, t3, logz)


def kernel(Y, pred_log_prob, batch_idx):
    flat = Y[:, 0] + Y[:, 1] * 16
    hist_parts, t = _sc_kernel()(flat, batch_idx)
    xt = pred_log_prob.transpose(1, 2, 0).reshape(NB, BATCH)
    marg_t, logz = _tc1_call(xt)
    t3 = t.reshape(GRID, 1, C_BLK)
    thrf, lc, loss = _tc2_call(xt, hist_parts, t3, logz)
    marginals = marg_t.reshape(2, KK, BATCH).transpose(2, 0, 1)
    return (loss[0, 0], marginals, thrf.reshape(KK, KK), lc)


# speedup vs baseline: 3.2737x; 1.0008x over previous
import functools

import jax
import jax.numpy as jnp
from jax import lax
from jax.experimental import pallas as pl
from jax.experimental.pallas import tpu as pltpu
from jax.experimental.pallas import tpu_sc as plsc

KK = 16
NB = 256
NROWS = 1_000_000
BATCH = 4096
NW = 32
ROWS_MAIN = 31_248
ROWS_TAIL = NROWS - NW * ROWS_MAIN
ITERS_MAIN = ROWS_MAIN // 16
ITERS_TAIL = ROWS_TAIL // 16
CHUNK_W = ROWS_MAIN + ROWS_TAIL
B_PER_W = BATCH // NW


def _sc_histogram_kernel():
    mesh = plsc.VectorSubcoreMesh(core_axis_name="c", subcore_axis_name="s",
                                  num_cores=2, num_subcores=16)

    @functools.partial(
        pl.kernel,
        out_type=[
            jax.ShapeDtypeStruct((NW, NB), jnp.int32),
            jax.ShapeDtypeStruct((BATCH,), jnp.int32),
        ],
        mesh=mesh,
        compiler_params=pltpu.CompilerParams(needs_layout_passes=False),
        scratch_types=[
            pltpu.VMEM((CHUNK_W,), jnp.int32),
            pltpu.VMEM((16, NB), jnp.int32),
            pltpu.VMEM((NB,), jnp.int32),
            pltpu.VMEM((B_PER_W,), jnp.int32),
            pltpu.VMEM((B_PER_W,), jnp.int32),
            pltpu.VMEM((B_PER_W,), jnp.int32),
            pltpu.SemaphoreType.DMA,
            pltpu.SemaphoreType.DMA,
        ],
    )
    def sc_k(flat_hbm, bidx_hbm, hist_hbm, t_hbm,
             chunk, hist, histrow, bidx_v, g_v, t_v, sem_c, sem_g):
        wid = lax.axis_index("c") * 16 + lax.axis_index("s")
        lane = lax.iota(jnp.int32, 16)
        ones = jnp.full((16,), 1, jnp.int32)
        zeros = jnp.zeros((16,), jnp.int32)

        base = wid * ROWS_MAIN
        cp = pltpu.async_copy(flat_hbm.at[pl.ds(base, ROWS_MAIN)],
                              chunk.at[pl.ds(0, ROWS_MAIN)], sem_c)

        pltpu.sync_copy(bidx_hbm.at[pl.ds(wid * B_PER_W, B_PER_W)], bidx_v)
        ga = pltpu.async_copy(flat_hbm.at[bidx_v], g_v, sem_g)

        for l in range(16):
            for j in range(NB // 16):
                hist[l, pl.ds(j * 16, 16)] = zeros

        ga.wait()
        for j in range(B_PER_W // 16):
            sl = pl.ds(j * 16, 16)
            f = g_v[sl]
            t_v[sl] = (f & 15) * 16 + (f >> 4)
        pltpu.sync_copy(t_v, t_hbm.at[pl.ds(wid * B_PER_W, B_PER_W)])

        @pl.when(wid == NW - 1)
        def _():
            pltpu.sync_copy(flat_hbm.at[pl.ds(NW * ROWS_MAIN, ROWS_TAIL)],
                            chunk.at[pl.ds(ROWS_MAIN, ROWS_TAIL)])

        cp.wait()

        def hist_body(i, carry):
            flat = chunk[pl.ds(i * 16, 16)]
            plsc.addupdate_scatter(hist, [lane, flat], ones)
            return carry

        lax.fori_loop(0, ITERS_MAIN, hist_body, 0, unroll=8)

        @pl.when(wid == NW - 1)
        def _():
            for i in range(ITERS_MAIN, ITERS_MAIN + ITERS_TAIL):
                flat = chunk[pl.ds(i * 16, 16)]
                plsc.addupdate_scatter(hist, [lane, flat], ones)

        for j in range(NB // 16):
            acc = hist[0, pl.ds(j * 16, 16)]
            for l in range(1, 16):
                acc = acc + hist[l, pl.ds(j * 16, 16)]
            histrow[pl.ds(j * 16, 16)] = acc
        pltpu.sync_copy(histrow, hist_hbm.at[wid])

    return sc_k


_SC_KERNEL_CACHE = []


def _sc_kernel():
    if not _SC_KERNEL_CACHE:
        _SC_KERNEL_CACHE.append(_sc_histogram_kernel())
    return _SC_KERNEL_CACHE[0]


C_BLK = 512
GRID = BATCH // C_BLK


def _tc1_body(x_ref, marg_ref, logz_ref):
    x = x_ref[...]
    m = jnp.max(x, axis=0, keepdims=True)
    e = jnp.exp(x - m)
    s = jnp.sum(e, axis=0, keepdims=True)
    en = e / s

    ri = lax.broadcasted_iota(jnp.int32, (KK, NB), 0)
    ci = lax.broadcasted_iota(jnp.int32, (KK, NB), 1)
    m0 = ((ci // KK) == ri).astype(jnp.float32)
    m1 = ((ci % KK) == ri).astype(jnp.float32)
    marg_ref[0:KK, :] = jnp.dot(m0, en, preferred_element_type=jnp.float32)
    marg_ref[KK:2 * KK, :] = jnp.dot(m1, en, preferred_element_type=jnp.float32)
    logz_ref[...] = m + jnp.log(s)


def _tc1_call(xt):
    return pl.pallas_call(
        _tc1_body,
        grid=(GRID,),
        in_specs=[pl.BlockSpec((NB, C_BLK), lambda g: (0, g))],
        out_specs=[
            pl.BlockSpec((2 * KK, C_BLK), lambda g: (0, g)),
            pl.BlockSpec((1, C_BLK), lambda g: (0, g)),
        ],
        out_shape=[
            jax.ShapeDtypeStruct((2 * KK, BATCH), jnp.float32),
            jax.ShapeDtypeStruct((1, BATCH), jnp.float32),
        ],
    )(xt)


def _tc2_body(x_ref, h_ref, t_ref, logz_ref, thr_ref, lc_ref, loss_ref):
    g = pl.program_id(0)
    jf = jnp.sum(h_ref[...], axis=0, keepdims=True)
    thrf = lax.rsqrt(jnp.sqrt(jf.astype(jnp.float32) + 1.0))

    @pl.when(g == 0)
    def _():
        thr_ref[...] = thrf
        ri = lax.broadcasted_iota(jnp.int32, (KK, NB), 0)
        ci = lax.broadcasted_iota(jnp.int32, (KK, NB), 1)
        jfb = jnp.broadcast_to(jf, (KK, NB))
        lc_ref[0, :] = jnp.sum(jnp.where((ci % KK) == ri, jfb, 0), axis=1)
        lc_ref[1, :] = jnp.sum(jnp.where((ci // KK) == ri, jfb, 0), axis=1)
        loss_ref[...] = jnp.zeros((1, 1), jnp.float32)

    x = x_ref[...]
    t = t_ref[0, 0, :].reshape(1, C_BLK)
    rows = lax.broadcasted_iota(jnp.int32, (NB, C_BLK), 0)
    mask = rows == t
    p = jnp.sum(jnp.where(mask, x, 0.0), axis=0, keepdims=True)
    w = jnp.dot(thrf, mask.astype(jnp.float32),
                preferred_element_type=jnp.float32)
    nll = logz_ref[...] - p
    part = jnp.sum(nll * w) * (1.0 / BATCH)
    loss_ref[...] += jnp.full((1, 1), 1.0, jnp.float32) * part


def _tc2_call(xt, hist_parts, t3, logz):
    return pl.pallas_call(
        _tc2_body,
        grid=(GRID,),
        in_specs=[
            pl.BlockSpec((NB, C_BLK), lambda g: (0, g)),
            pl.BlockSpec((NW, NB), lambda g: (0, 0)),
            pl.BlockSpec((1, 1, C_BLK), lambda g: (g, 0, 0)),
            pl.BlockSpec((1, C_BLK), lambda g: (0, g)),
        ],
        out_specs=[
            pl.BlockSpec((1, NB), lambda g: (0, 0)),
            pl.BlockSpec((2, KK), lambda g: (0, 0)),
            pl.BlockSpec((1, 1), lambda g: (0, 0)),
        ],
        out_shape=[
            jax.ShapeDtypeStruct((1, NB), jnp.float32),
            jax.ShapeDtypeStruct((2, KK), jnp.int32),
            jax.ShapeDtypeStruct((1, 1), jnp.float32),
        ],
    )(xt, hist_parts, t3, logz)


def kernel(Y, pred_log_prob, batch_idx):
    flat = Y[:, 0] + Y[:, 1] * 16
    hist_parts, t = _sc_kernel()(flat, batch_idx)
    xt = pred_log_prob.transpose(1, 2, 0).reshape(NB, BATCH)
    marg_t, logz = _tc1_call(xt)
    t3 = t.reshape(GRID, 1, C_BLK)
    thrf, lc, loss = _tc2_call(xt, hist_parts, t3, logz)
    marginals = marg_t.reshape(2, KK, BATCH).transpose(2, 0, 1)
    return (loss[0, 0], marginals, thrf.reshape(KK, KK), lc)
